# Initial kernel scaffold; baseline (speedup 1.0000x reference)
import jax
import jax.numpy as jnp
from jax.experimental import pallas as pl


def kernel(x, t, past_x, past_t, covariates_xt, z_grid, x_grid, t_grid, beta, alpha, sigma, omega):
    raise NotImplementedError("write your pallas kernel here")


# speedup vs baseline: 2.9245x; 2.9245x over previous
import jax
import jax.numpy as jnp
from jax.experimental import pallas as pl
from jax.experimental.pallas import tpu as pltpu

TWO_PI = 6.283185307179586
EPS = 1e-6


def _hawkes_body(x_ref, t_ref, px_ref, pt_ref, cov_ref, z_ref, xg_ref,
                 tg_ref, beta_ref, scal_ref, log_ref, cross_ref, base_ref):
    alpha = scal_ref[0, 0]
    sigma = scal_ref[0, 1]
    omega = scal_ref[0, 2]
    inv2s2 = -0.5 / (sigma * sigma)
    snorm = 1.0 / (TWO_PI * sigma * sigma)

    x0 = x_ref[:, 0:1]
    x1 = x_ref[:, 1:2]
    tb = t_ref[:, :]

    d0 = x0 - px_ref[0]
    d1 = x1 - px_ref[1]
    td = tb - pt_ref[:, :]
    expo = (d0 * d0 + d1 * d1) * inv2s2 - omega * td
    exc = jnp.where(td > 0.0, jnp.exp(expo), 0.0)
    exc_sum = exc.sum(axis=1, keepdims=True) * (alpha * snorm * omega)

    mu = jnp.dot(cov_ref[:, :], beta_ref[:, :],
                 preferred_element_type=jnp.float32)
    lam = jnp.maximum(mu, EPS) + exc_sum
    log_ref[:, :] = jnp.log(lam + EPS)

    g0 = x0 - xg_ref[0:1, :]
    g1 = x1 - xg_ref[1:2, :]
    s_sum = jnp.exp((g0 * g0 + g1 * g1) * inv2s2).sum(axis=1, keepdims=True)
    dtg = tg_ref[0:1, :] - tb
    w = jnp.where(dtg > 0.0, jnp.exp(-omega * dtg), 0.0)
    w_sum = w.sum(axis=1, keepdims=True)
    cross = (s_sum * w_sum).sum(axis=0, keepdims=True)
    cross_ref[0] = cross * (alpha * snorm * omega)

    zb = jnp.dot(z_ref[:, :], beta_ref[:, :],
                 preferred_element_type=jnp.float32)
    base_ref[0] = jnp.maximum(zb, EPS).sum(axis=0, keepdims=True)


def kernel(x, t, past_x, past_t, covariates_xt, z_grid, x_grid, t_grid,
           beta, alpha, sigma, omega):
    N, M = past_t.shape
    T, G, D = z_grid.shape
    TG = T * G
    Bn = 128
    NB = N // Bn
    Zc = TG // NB

    px = jnp.moveaxis(past_x, 2, 0)
    t2 = t[:, None]
    z2 = z_grid.reshape(TG, D)
    xg = x_grid.T
    tg2 = t_grid[None, :]
    beta2 = beta[:, None]
    scal = jnp.stack([alpha, sigma, omega]).astype(jnp.float32)[None, :]

    log_int, cross, base = pl.pallas_call(
        _hawkes_body,
        grid=(NB,),
        in_specs=[
            pl.BlockSpec((Bn, 2), lambda i: (i, 0)),
            pl.BlockSpec((Bn, 1), lambda i: (i, 0)),
            pl.BlockSpec((2, Bn, M), lambda i: (0, i, 0)),
            pl.BlockSpec((Bn, M), lambda i: (i, 0)),
            pl.BlockSpec((Bn, D), lambda i: (i, 0)),
            pl.BlockSpec((Zc, D), lambda i: (i, 0)),
            pl.BlockSpec((2, G), lambda i: (0, 0)),
            pl.BlockSpec((1, T), lambda i: (0, 0)),
            pl.BlockSpec((D, 1), lambda i: (0, 0)),
            pl.BlockSpec((1, 3), lambda i: (0, 0)),
        ],
        out_specs=[
            pl.BlockSpec((Bn, 1), lambda i: (i, 0)),
            pl.BlockSpec((1, 1, 1), lambda i: (i, 0, 0)),
            pl.BlockSpec((1, 1, 1), lambda i: (i, 0, 0)),
        ],
        out_shape=[
            jax.ShapeDtypeStruct((N, 1), jnp.float32),
            jax.ShapeDtypeStruct((NB, 1, 1), jnp.float32),
            jax.ShapeDtypeStruct((NB, 1, 1), jnp.float32),
        ],
        compiler_params=pltpu.CompilerParams(
            dimension_semantics=("parallel",),
        ),
        name="hawkes_fused",
    )(x, t2, px, past_t, covariates_xt, z2, xg, tg2, beta2, scal)

    dxdy = 1.0 / G
    dt_step = t_grid[1] - t_grid[0]
    integral = (base.sum() + cross.sum()) * (dxdy * dt_step)
    return jnp.concatenate([log_int[:, 0], integral[None]])
